# Initial kernel scaffold; baseline (speedup 1.0000x reference)
import jax
import jax.numpy as jnp
from jax.experimental import pallas as pl


def kernel(grid_input, params, mesh_spatial, g2m_edge_in, m2m_edge_in, m2g_edge_in, g2m_src, g2m_dst, m2m_src, m2m_dst, m2g_src, m2g_dst):
    raise NotImplementedError("write your pallas kernel here")


# speedup vs baseline: 2.5397x; 2.5397x over previous
import functools

import jax
import jax.numpy as jnp
from jax import lax
from jax.experimental import pallas as pl
from jax.experimental.pallas import tpu as pltpu
from jax.experimental.pallas import tpu_sc as plsc

C = 31
H = 121
W = 240
HID = 128
GRID_N = H * W
MESH_N = 10242
NC = 2
NS = 16
NW = NC * NS

G2M_P = 118784
M2M_P = 81920
M2G_P = 90112

NACC_MESH = 10368
TRASH_MESH = MESH_N
GRID_HALF = GRID_N // 2
NACC_GRID = 14592


def _pad_rows(a, n):
    return jnp.pad(a, ((0, n - a.shape[0]), (0, 0)))


def _pad_idx(a, n, val):
    return jnp.pad(a.astype(jnp.int32), (0, n - a.shape[0]), constant_values=val)


def _mlp(p, parts, residual=None, bm=1024):
    n = len(parts)
    M = parts[0][0].shape[0]
    W1 = p["W1"]
    dout = W1.shape[1]
    has_ln = "g" in p
    has_res = residual is not None
    b0 = p["b0"].reshape(1, HID)
    b1 = p["b1"].reshape(1, dout)

    def body(*refs):
        xs = refs[0:n]
        ws = refs[n:2 * n]
        b0r, w1r, b1r = refs[2 * n:2 * n + 3]
        i = 2 * n + 3
        if has_ln:
            gr, bnr = refs[i], refs[i + 1]
            i += 2
        if has_res:
            rr = refs[i]
            i += 1
        outr = refs[i]
        h = jnp.dot(xs[0][...], ws[0][...], preferred_element_type=jnp.float32)
        for j in range(1, n):
            h = h + jnp.dot(xs[j][...], ws[j][...],
                            preferred_element_type=jnp.float32)
        h = h + b0r[...]
        h = h * jax.nn.sigmoid(h)
        y = jnp.dot(h, w1r[...], preferred_element_type=jnp.float32) + b1r[...]
        if has_ln:
            m = jnp.mean(y, axis=-1, keepdims=True)
            v = jnp.mean((y - m) * (y - m), axis=-1, keepdims=True)
            y = (y - m) * lax.rsqrt(v + 1e-5) * gr[...] + bnr[...]
        if has_res:
            y = y + rr[...]
        outr[...] = y

    in_specs = []
    args = []
    for x, w0 in parts:
        in_specs.append(pl.BlockSpec((bm, x.shape[1]), lambda i: (i, 0)))
        args.append(x)
    for x, w0 in parts:
        in_specs.append(pl.BlockSpec(w0.shape, lambda i: (0, 0)))
        args.append(w0)
    in_specs.append(pl.BlockSpec((1, HID), lambda i: (0, 0)))
    args.append(b0)
    in_specs.append(pl.BlockSpec(W1.shape, lambda i: (0, 0)))
    args.append(W1)
    in_specs.append(pl.BlockSpec((1, dout), lambda i: (0, 0)))
    args.append(b1)
    if has_ln:
        in_specs.append(pl.BlockSpec((1, dout), lambda i: (0, 0)))
        args.append(p["g"].reshape(1, dout))
        in_specs.append(pl.BlockSpec((1, dout), lambda i: (0, 0)))
        args.append(p["bn"].reshape(1, dout))
    if has_res:
        in_specs.append(pl.BlockSpec((bm, dout), lambda i: (i, 0)))
        args.append(residual)

    return pl.pallas_call(
        body,
        grid=(pl.cdiv(M, bm),),
        in_specs=in_specs,
        out_specs=pl.BlockSpec((bm, dout), lambda i: (i, 0)),
        out_shape=jax.ShapeDtypeStruct((M, dout), jnp.float32),
    )(*args)


def _sc_gather_pair(ta, ia, tb, ib):
    E = ia.shape[0]
    epw = E // NW
    nch = epw // 128
    mesh = plsc.VectorSubcoreMesh(core_axis_name="c", subcore_axis_name="s")

    @functools.partial(
        pl.kernel,
        mesh=mesh,
        out_type=(jax.ShapeDtypeStruct((E, HID), jnp.float32),
                  jax.ShapeDtypeStruct((E, HID), jnp.float32)),
        scratch_types=[
            pltpu.VMEM((epw,), jnp.int32),
            pltpu.VMEM((epw,), jnp.int32),
            pltpu.VMEM((128, HID), jnp.float32),
            pltpu.VMEM((128, HID), jnp.float32),
            pltpu.SemaphoreType.DMA,
            pltpu.SemaphoreType.DMA,
        ],
    )
    def k(ta_h, ia_h, tb_h, ib_h, oa_h, ob_h, iva, ivb, ra, rb, sa, sb):
        c = lax.axis_index("c")
        s = lax.axis_index("s")
        base = (s * NC + c) * epw
        pltpu.sync_copy(ia_h.at[pl.ds(base, epw)], iva)
        pltpu.sync_copy(ib_h.at[pl.ds(base, epw)], ivb)

        def step(j, carry):
            o = j * 128
            ca = pltpu.async_copy(ta_h.at[iva.at[pl.ds(o, 128)]], ra, sa)
            cb = pltpu.async_copy(tb_h.at[ivb.at[pl.ds(o, 128)]], rb, sb)
            ca.wait()
            cb.wait()
            pltpu.sync_copy(ra, oa_h.at[pl.ds(base + o, 128)])
            pltpu.sync_copy(rb, ob_h.at[pl.ds(base + o, 128)])
            return carry

        lax.fori_loop(0, nch, step, 0)

    return k(ta, ia, tb, ib)


def _sc_segsum(vals, dst, nacc, span, scan_all, ck=128):
    E = dst.shape[0]
    ept = E // (NS if scan_all else NW)
    ncpt = ept // ck
    rpt = nacc // NS
    zeros = jnp.zeros((nacc, HID), jnp.float32)
    mesh = plsc.VectorSubcoreMesh(core_axis_name="c", subcore_axis_name="s")

    @functools.partial(
        pl.kernel,
        mesh=mesh,
        out_type=jax.ShapeDtypeStruct((NC, nacc, HID), jnp.float32),
        scratch_types=[
            pltpu.VMEM_SHARED((nacc, HID), jnp.float32),
            pltpu.VMEM((ck,), jnp.int32),
            pltpu.VMEM((ck, HID), jnp.float32),
        ],
    )
    def k(vals_h, dst_h, zeros_h, out_h, acc, ibuf, vbuf):
        c = lax.axis_index("c")
        s = lax.axis_index("s")
        r0 = s * rpt
        pltpu.sync_copy(zeros_h.at[pl.ds(r0, rpt)], acc.at[pl.ds(r0, rpt)])
        plsc.subcore_barrier()
        if scan_all:
            eoff = s * ept
        else:
            eoff = (c * NS + s) * ept

        def step(j, carry):
            o = eoff + j * ck
            pltpu.sync_copy(dst_h.at[pl.ds(o, ck)], ibuf)

            def fix(kk, carry2):
                v = ibuf[pl.ds(kk * 16, 16)]
                if scan_all:
                    l = v - c * span
                    ok = (l >= 0) & (l < span)
                    v = jnp.where(ok, l, span)
                else:
                    v = jnp.where(v >= 0, v, span)
                ibuf[pl.ds(kk * 16, 16)] = v
                return carry2

            lax.fori_loop(0, ck // 16, fix, 0)
            pltpu.sync_copy(vals_h.at[pl.ds(o, ck)], vbuf)
            pltpu.sync_copy(vbuf, acc.at[ibuf], add=True)
            return carry

        lax.fori_loop(0, ncpt, step, 0)
        plsc.subcore_barrier()
        pltpu.sync_copy(acc.at[pl.ds(r0, rpt)], out_h.at[c, pl.ds(r0, rpt)])

    return k(vals, dst, zeros)


def kernel(grid_input, params, mesh_spatial, g2m_edge_in, m2m_edge_in,
           m2g_edge_in, g2m_src, g2m_dst, m2m_src, m2m_dst, m2g_src, m2g_dst):
    p = params
    x = grid_input.reshape(C, GRID_N).T

    g2m_src_g = _pad_idx(g2m_src, G2M_P, 0)
    g2m_dst_g = _pad_idx(g2m_dst, G2M_P, 0)
    g2m_dst_s = _pad_idx(g2m_dst, G2M_P, -1)
    m2m_src_g = _pad_idx(m2m_src, M2M_P, 0)
    m2m_dst_g = _pad_idx(m2m_dst, M2M_P, 0)
    m2m_dst_s = _pad_idx(m2m_dst, M2M_P, -1)
    m2g_src_g = _pad_idx(m2g_src, M2G_P, 0)
    m2g_dst_g = _pad_idx(m2g_dst, M2G_P, 0)
    m2g_dst_s = _pad_idx(m2g_dst, M2G_P, -1)

    grid_emb = _mlp(p["emb_grid"], [(x, p["emb_grid"]["W0"])])
    mesh_emb = _mlp(p["emb_mesh"], [(mesh_spatial, p["emb_mesh"]["W0"])])
    g2m_e = _mlp(p["emb_g2m_e"],
                 [(_pad_rows(g2m_edge_in, G2M_P), p["emb_g2m_e"]["W0"])])
    m2m_e = _mlp(p["emb_m2m_e"],
                 [(_pad_rows(m2m_edge_in, M2M_P), p["emb_m2m_e"]["W0"])])
    m2g_e = _mlp(p["emb_m2g_e"],
                 [(_pad_rows(m2g_edge_in, M2G_P), p["emb_m2g_e"]["W0"])])

    gs, gd = _sc_gather_pair(grid_emb, g2m_src_g, mesh_emb, g2m_dst_g)
    w0 = p["enc_edge"]["W0"]
    ef = _mlp(p["enc_edge"], [(g2m_e, w0[:HID]), (gs, w0[HID:2 * HID]),
                              (gd, w0[2 * HID:])])
    agg = _sc_segsum(ef, g2m_dst_s, NACC_MESH, TRASH_MESH, scan_all=False)
    w0 = p["enc_dst"]["W0"]
    mesh_h = _mlp(p["enc_dst"],
                  [(mesh_emb, w0[:HID]), (agg[0, :MESH_N], w0[HID:]),
                   (agg[1, :MESH_N], w0[HID:])],
                  residual=mesh_emb)
    grid_h = _mlp(p["enc_src"], [(grid_emb, p["enc_src"]["W0"])],
                  residual=grid_emb)

    e = m2m_e
    n = mesh_h
    for lp in p["proc"]:
        ns, nd = _sc_gather_pair(n, m2m_src_g, n, m2m_dst_g)
        w0 = lp["edge"]["W0"]
        e = _mlp(lp["edge"], [(e, w0[:HID]), (ns, w0[HID:2 * HID]),
                              (nd, w0[2 * HID:])], residual=e)
        agg = _sc_segsum(e, m2m_dst_s, NACC_MESH, TRASH_MESH, scan_all=False)
        w0 = lp["node"]["W0"]
        n = _mlp(lp["node"],
                 [(n, w0[:HID]), (agg[0, :MESH_N], w0[HID:]),
                  (agg[1, :MESH_N], w0[HID:])],
                 residual=n)

    ms, gh = _sc_gather_pair(n, m2g_src_g, grid_h, m2g_dst_g)
    w0 = p["dec_edge"]["W0"]
    ef = _mlp(p["dec_edge"], [(m2g_e, w0[:HID]), (ms, w0[HID:2 * HID]),
                              (gh, w0[2 * HID:])])
    gagg = _sc_segsum(ef, m2g_dst_s, NACC_GRID, GRID_HALF, scan_all=True,
                      ck=64)
    agg_grid = jnp.concatenate([gagg[0, :GRID_HALF], gagg[1, :GRID_HALF]], 0)
    w0 = p["dec_node"]["W0"]
    grid_out = _mlp(p["dec_node"], [(grid_h, w0[:HID]), (agg_grid, w0[HID:])],
                    residual=grid_h)
    out = _mlp(p["finale"], [(grid_out, p["finale"]["W0"])])
    return out.T.reshape(C, H, W)
